# baseline (speedup 1.0000x reference)
import functools

import jax
import jax.numpy as jnp
import numpy as np
from jax import lax
from jax.experimental import pallas as pl
from jax.experimental.pallas import tpu as pltpu
from jax.experimental.pallas import tpu_sc as plsc

NUM_EXPERTS = 16
TOP_K = 2
D_OUT = 64
_EPS = float(np.finfo(np.float64).eps)

_SC_INFO = plsc.get_sparse_core_info()
_NL = _SC_INFO.num_lanes
_NW = _SC_INFO.num_cores * _SC_INFO.num_subcores


def _gate_body(gsT_hbm, gatesT_hbm, imp_hbm, load_hbm,
               gin, gout, impv, loadv):
    wid = lax.axis_index("s") * _SC_INFO.num_cores + lax.axis_index("c")
    n_tok = gin.shape[1]
    base = wid * n_tok
    pltpu.sync_copy(gsT_hbm.at[:, pl.ds(base, n_tok)], gin)

    for e in range(NUM_EXPERTS):
        impv[e] = jnp.zeros((_NL,), jnp.float32)
        loadv[e] = jnp.zeros((_NL,), jnp.float32)

    ninf = jnp.full((_NL,), -jnp.inf, jnp.float32)
    for j in range(n_tok // _NL):
        sl = pl.ds(j * _NL, _NL)
        g = [gin[e, sl] for e in range(NUM_EXPERTS)]
        m1 = g[0]
        i1 = jnp.zeros((_NL,), jnp.int32)
        for e in range(1, NUM_EXPERTS):
            gt = g[e] > m1
            m1 = jnp.where(gt, g[e], m1)
            i1 = jnp.where(gt, e, i1)
        m2 = ninf
        i2 = jnp.zeros((_NL,), jnp.int32)
        for e in range(NUM_EXPERTS):
            cand = jnp.where(i1 == e, ninf, g[e])
            gt = cand > m2
            m2 = jnp.where(gt, cand, m2)
            i2 = jnp.where(gt, e, i2)
        e2 = jnp.exp(m2 - m1)
        denom = 1.0 + e2
        g1v = 1.0 / denom
        g2v = e2 / denom
        for e in range(NUM_EXPERTS):
            ge = jnp.where(i1 == e, g1v, jnp.where(i2 == e, g2v, 0.0))
            gout[e, sl] = ge
            impv[e] = impv[e] + ge
            loadv[e] = loadv[e] + jnp.where(ge > 0.0, 1.0, 0.0)

    pltpu.sync_copy(gout, gatesT_hbm.at[:, pl.ds(base, n_tok)])
    pltpu.sync_copy(impv, imp_hbm.at[wid])
    pltpu.sync_copy(loadv, load_hbm.at[wid])


def _sc_gating(gsT):
    B = gsT.shape[1]
    n_tok = B // _NW
    mesh = plsc.VectorSubcoreMesh(core_axis_name="c", subcore_axis_name="s")
    return pl.kernel(
        _gate_body,
        mesh=mesh,
        out_type=[
            jax.ShapeDtypeStruct((NUM_EXPERTS, B), jnp.float32),
            jax.ShapeDtypeStruct((_NW, NUM_EXPERTS, _NL), jnp.float32),
            jax.ShapeDtypeStruct((_NW, NUM_EXPERTS, _NL), jnp.float32),
        ],
        scratch_types=[
            pltpu.VMEM((NUM_EXPERTS, n_tok), jnp.float32),
            pltpu.VMEM((NUM_EXPERTS, n_tok), jnp.float32),
            pltpu.VMEM((NUM_EXPERTS, _NL), jnp.float32),
            pltpu.VMEM((NUM_EXPERTS, _NL), jnp.float32),
        ],
    )(gsT)


def _moe_body(x_ref, g_ref, w_ref, b_ref, bd_ref, exp_ref, sel_ref,
              imp_ref, load_ref, y_ref, loss_ref):
    step = pl.program_id(0)
    nsteps = pl.num_programs(0)

    xb = x_ref[...].astype(jnp.bfloat16)
    logits = (
        jnp.dot(xb, w_ref[...], preferred_element_type=jnp.float32) + b_ref[...]
    )
    t = jnp.exp(logits).astype(jnp.bfloat16)
    s = jnp.dot(t, bd_ref[...], preferred_element_type=jnp.float32)
    gates = g_ref[...].T
    q = (gates / s).astype(jnp.bfloat16)
    qb = jnp.dot(q, exp_ref[...], preferred_element_type=jnp.float32)
    combined = jnp.dot(t * qb.astype(jnp.bfloat16), sel_ref[...],
                       preferred_element_type=jnp.float32)
    y_ref[...] = jnp.log(jnp.where(combined == 0.0, _EPS, combined))

    @pl.when(step == nsteps - 1)
    def _():
        def cv_sq(v):
            mean = jnp.sum(v) / NUM_EXPERTS
            var = jnp.sum((v - mean) ** 2) / (NUM_EXPERTS - 1)
            return var / (mean * mean + 1e-10)

        imp = jnp.sum(jnp.sum(imp_ref[...], axis=0), axis=-1, keepdims=True)
        load = jnp.sum(jnp.sum(load_ref[...], axis=0), axis=-1, keepdims=True)
        loss = cv_sq(imp) + cv_sq(load)
        loss_ref[...] = jnp.broadcast_to(loss, (1, 1))


@functools.partial(jax.jit, static_argnames=("block_b",))
def _moe_fused(x2, gsT, wt, bflat, block_b=1024):
    B = x2.shape[0]
    d_in = x2.shape[1]
    EH = NUM_EXPERTS * D_OUT
    grid = (B // block_b,)

    gatesT, imp_parts, load_parts = _sc_gating(gsT)

    lane = np.arange(EH)
    bd = (lane[:, None] // D_OUT == np.arange(NUM_EXPERTS)[None, :])
    bd = bd.astype(jnp.bfloat16)
    expand = bd.T.copy()
    sel = (lane[:, None] % D_OUT == np.arange(D_OUT)[None, :]).astype(jnp.bfloat16)

    y, loss = pl.pallas_call(
        _moe_body,
        grid=grid,
        in_specs=[
            pl.BlockSpec((block_b, d_in), lambda i: (i, 0)),
            pl.BlockSpec((NUM_EXPERTS, block_b), lambda i: (0, i)),
            pl.BlockSpec((d_in, EH), lambda i: (0, 0)),
            pl.BlockSpec((1, EH), lambda i: (0, 0)),
            pl.BlockSpec((EH, NUM_EXPERTS), lambda i: (0, 0)),
            pl.BlockSpec((NUM_EXPERTS, EH), lambda i: (0, 0)),
            pl.BlockSpec((EH, D_OUT), lambda i: (0, 0)),
            pl.BlockSpec((_NW, NUM_EXPERTS, _NL), lambda i: (0, 0, 0)),
            pl.BlockSpec((_NW, NUM_EXPERTS, _NL), lambda i: (0, 0, 0)),
        ],
        out_specs=[
            pl.BlockSpec((block_b, D_OUT), lambda i: (i, 0)),
            pl.BlockSpec((1, 1), lambda i: (0, 0)),
        ],
        out_shape=[
            jax.ShapeDtypeStruct((B, D_OUT), jnp.float32),
            jax.ShapeDtypeStruct((1, 1), jnp.float32),
        ],
    )(x2, gatesT, wt, bflat, jnp.asarray(bd), jnp.asarray(expand),
      jnp.asarray(sel), imp_parts, load_parts)
    return y, loss[0, 0]


def kernel(x, gate_scores, W, b):
    Bx = x.shape[0]
    x2 = x.reshape(Bx, -1)
    wt = W.transpose(1, 0, 2).reshape(x2.shape[1], NUM_EXPERTS * D_OUT)
    wt = wt.astype(jnp.bfloat16)
    bflat = b.reshape(1, NUM_EXPERTS * D_OUT)
    return _moe_fused(x2, gate_scores.T, wt, bflat)
